# Initial kernel scaffold; baseline (speedup 1.0000x reference)
import jax
import jax.numpy as jnp
from jax.experimental import pallas as pl


def kernel(x, edge_index, batch, Wl1, Wr1, a1, b1, Wl2, Wr2, a2, b2, Wl3, Wr3, a3, b3, Wc1, bc1, Wc2, bc2, Wc3, bc3, Wc4, bc4, Wc5, bc5):
    raise NotImplementedError("write your pallas kernel here")


# speedup vs baseline: 9.1587x; 9.1587x over previous
import functools

import jax
import jax.numpy as jnp
import numpy as np
from jax import lax
from jax.experimental import pallas as pl
from jax.experimental.pallas import tpu as pltpu
from jax.experimental.pallas import tpu_sc as plsc

N_NODES = 10000
N_PAD = 10240
N_EDGES = 320000
N_GRAPHS = 64

NC, NS, LANES = 2, 16, 16
NW = NC * NS
EPW = 10240
PAD_DST = 10239
CHUNK = 64
NCHUNK = EPW // CHUNK
ROWS_PER_SUB = N_PAD // NS
ZROWS = 128


def _edge_pass(D):
    mesh = plsc.VectorSubcoreMesh(core_axis_name="c", subcore_axis_name="s")

    @functools.partial(
        pl.kernel,
        out_type=(
            jax.ShapeDtypeStruct((NC, N_PAD, D), jnp.float32),
            jax.ShapeDtypeStruct((NC, N_PAD), jnp.float32),
        ),
        mesh=mesh,
        compiler_params=pltpu.CompilerParams(needs_layout_passes=False,
                                             use_tc_tiling_on_sc=False),
        scratch_types=(
            pltpu.VMEM_SHARED((N_PAD, D), jnp.float32),
            pltpu.VMEM_SHARED((N_PAD,), jnp.float32),
            pltpu.VMEM((CHUNK,), jnp.int32),
            pltpu.VMEM((CHUNK,), jnp.int32),
            pltpu.VMEM((CHUNK, D), jnp.float32),
            pltpu.VMEM((CHUNK, D), jnp.float32),
            pltpu.VMEM((CHUNK, D), jnp.float32),
            pltpu.VMEM((CHUNK,), jnp.float32),
            pltpu.VMEM((D,), jnp.float32),
            pltpu.SemaphoreType.DMA,
            pltpu.SemaphoreType.DMA,
        ),
    )
    def edge_kernel(xl_hbm, xr_hbm, a_hbm, src_hbm, dst_hbm, znum_hbm,
                    zden_hbm, num_out, den_out,
                    num_sh, den_sh, idx_s, idx_d, rows_s, rows_d, wvals,
                    wbuf, av, sem_s, sem_d):
        cid = lax.axis_index("c")
        sid = lax.axis_index("s")
        wid = sid * NC + cid
        base = sid * ROWS_PER_SUB
        nsl = pl.ds(base, ROWS_PER_SUB)

        pltpu.sync_copy(znum_hbm.at[nsl], num_sh.at[nsl])
        pltpu.sync_copy(zden_hbm.at[nsl], den_sh.at[nsl])

        pltpu.sync_copy(a_hbm, av)

        plsc.subcore_barrier()

        slope = np.float32(0.2)
        eidx0 = jnp.arange(LANES, dtype=jnp.int32)
        av_regs = [av[pl.ds(i * LANES, LANES)] for i in range(D // LANES)]

        def chunk_body(j, _):
            pltpu.sync_copy(src_hbm.at[wid, j], idx_s)
            pltpu.sync_copy(dst_hbm.at[wid, j], idx_d)
            cp_s = pltpu.async_copy(xl_hbm.at[idx_s], rows_s, sem_s)
            cp_d = pltpu.async_copy(xr_hbm.at[idx_d], rows_d, sem_d)
            cp_s.wait()
            cp_d.wait()

            def group_body(g, _):
                lvec = jnp.zeros((LANES,), jnp.float32)
                for i in range(LANES):
                    e = g * LANES + i
                    acc = jnp.zeros((LANES,), jnp.float32)
                    for k in range(D // LANES):
                        sl = pl.ds(k * LANES, LANES)
                        v = rows_s[e, sl] + rows_d[e, sl]
                        acc = acc + jnp.maximum(v, v * slope) * av_regs[k]
                    lvec = jnp.where(eidx0 == i, jnp.sum(acc), lvec)
                wv = jnp.exp(lvec)
                wbuf[pl.ds(g * LANES, LANES)] = wv
                for i in range(LANES):
                    e = g * LANES + i
                    for k in range(D // LANES):
                        sl = pl.ds(k * LANES, LANES)
                        wvals[e, sl] = rows_s[e, sl] * wv[i]
                return 0
            lax.fori_loop(0, CHUNK // LANES, group_body, 0)

            pltpu.sync_copy(wvals, num_sh.at[idx_d], add=True)
            pltpu.sync_copy(wbuf, den_sh.at[idx_d], add=True)
            return 0
        lax.fori_loop(0, NCHUNK, chunk_body, 0)

        plsc.subcore_barrier()

        pltpu.sync_copy(num_sh.at[nsl], num_out.at[cid].at[nsl])
        pltpu.sync_copy(den_sh.at[nsl], den_out.at[cid].at[nsl])

    return edge_kernel


_EDGE_KERNELS = {d: _edge_pass(d) for d in (32, 64, 128)}


def _pre_body(x_ref, wl_ref, wr_ref, xl_ref, xr_ref):
    x = x_ref[...]
    xl_ref[...] = jnp.dot(x, wl_ref[...], preferred_element_type=jnp.float32)
    xr_ref[...] = jnp.dot(x, wr_ref[...], preferred_element_type=jnp.float32)


def _mid_body(num_ref, den_ref, b_ref, wl_ref, wr_ref, xl_ref, xr_ref):
    num = num_ref[0, :N_NODES, :] + num_ref[1, :N_NODES, :]
    den = den_ref[0, :N_NODES, :] + den_ref[1, :N_NODES, :]
    h = num / (den + np.float32(1e-16)) + b_ref[...]
    h = jnp.maximum(h, h * np.float32(0.01))
    xl_ref[...] = jnp.dot(h, wl_ref[...], preferred_element_type=jnp.float32)
    xr_ref[...] = jnp.dot(h, wr_ref[...], preferred_element_type=jnp.float32)


def _final_body(num_ref, den_ref, b_ref, batch_ref,
                wc1_ref, bc1_ref, wc2_ref, bc2_ref, wc3_ref, bc3_ref,
                wc4_ref, bc4_ref, wc5_ref, bc5_ref, z_ref, g_ref):
    num = num_ref[0, :N_NODES, :] + num_ref[1, :N_NODES, :]
    den = den_ref[0, :N_NODES, :] + den_ref[1, :N_NODES, :]
    h = num / (den + np.float32(1e-16)) + b_ref[...]
    bvec = batch_ref[...]

    def pool_body(g, _):
        hm = jnp.where(bvec == g, h, -jnp.inf)
        g_ref[pl.ds(g, 1), :] = jnp.max(hm, axis=0, keepdims=True)
        return 0
    lax.fori_loop(0, N_GRAPHS, pool_body, 0)

    z = g_ref[...]
    z = jnp.maximum(jnp.dot(z, wc1_ref[...], preferred_element_type=jnp.float32) + bc1_ref[...], 0.0)
    z = jnp.maximum(jnp.dot(z, wc2_ref[...], preferred_element_type=jnp.float32) + bc2_ref[...], 0.0)
    z = jnp.maximum(jnp.dot(z, wc3_ref[...], preferred_element_type=jnp.float32) + bc3_ref[...], 0.0)
    z = jnp.maximum(jnp.dot(z, wc4_ref[...], preferred_element_type=jnp.float32) + bc4_ref[...], 0.0)
    z_ref[...] = jnp.dot(z, wc5_ref[...], preferred_element_type=jnp.float32) + bc5_ref[...]


def _pre_call(x, wl, wr):
    n, dout = x.shape[0], wl.shape[1]
    return pl.pallas_call(
        _pre_body,
        out_shape=(jax.ShapeDtypeStruct((n, dout), jnp.float32),
                   jax.ShapeDtypeStruct((n, dout), jnp.float32)),
    )(x, wl, wr)


def _mid_call(num, den, b, wl, wr):
    dout = wl.shape[1]
    return pl.pallas_call(
        _mid_body,
        out_shape=(jax.ShapeDtypeStruct((N_NODES, dout), jnp.float32),
                   jax.ShapeDtypeStruct((N_NODES, dout), jnp.float32)),
    )(num, den.reshape(NC, N_PAD, 1), b.reshape(1, -1), wl, wr)


def _final_call(num, den, b, batch, wc_bc):
    args = [num, den.reshape(NC, N_PAD, 1), b.reshape(1, -1),
            batch.reshape(N_NODES, 1)]
    for w, bc in wc_bc:
        args.extend([w, bc.reshape(1, -1)])
    return pl.pallas_call(
        _final_body,
        out_shape=jax.ShapeDtypeStruct((N_GRAPHS, 4), jnp.float32),
        scratch_shapes=[pltpu.VMEM((N_GRAPHS, 128), jnp.float32)],
    )(*args)


def kernel(x, edge_index, batch,
           Wl1, Wr1, a1, b1,
           Wl2, Wr2, a2, b2,
           Wl3, Wr3, a3, b3,
           Wc1, bc1, Wc2, bc2, Wc3, bc3, Wc4, bc4, Wc5, bc5):
    npad = NW * EPW - N_EDGES
    src3 = jnp.concatenate(
        [edge_index[0], jnp.zeros((npad,), jnp.int32)]).reshape(NW, NCHUNK, CHUNK)
    dst3 = jnp.concatenate(
        [edge_index[1], jnp.full((npad,), PAD_DST, jnp.int32)]).reshape(NW, NCHUNK, CHUNK)

    zden = jnp.zeros((N_PAD,), jnp.float32)

    xl, xr = _pre_call(x, Wl1, Wr1)
    num, den = _EDGE_KERNELS[32](xl, xr, a1, src3, dst3,
                                 jnp.zeros((N_PAD, 32), jnp.float32), zden)
    xl, xr = _mid_call(num, den, b1, Wl2, Wr2)
    num, den = _EDGE_KERNELS[64](xl, xr, a2, src3, dst3,
                                 jnp.zeros((N_PAD, 64), jnp.float32), zden)
    xl, xr = _mid_call(num, den, b2, Wl3, Wr3)
    num, den = _EDGE_KERNELS[128](xl, xr, a3, src3, dst3,
                                  jnp.zeros((N_PAD, 128), jnp.float32), zden)
    return _final_call(num, den, b3, batch,
                       [(Wc1, bc1), (Wc2, bc2), (Wc3, bc3), (Wc4, bc4), (Wc5, bc5)])


# speedup vs baseline: 19.8998x; 2.1728x over previous
import functools

import jax
import jax.numpy as jnp
import numpy as np
from jax import lax
from jax.experimental import pallas as pl
from jax.experimental.pallas import tpu as pltpu
from jax.experimental.pallas import tpu_sc as plsc

N_NODES = 10000
N_PAD = 10240
N_EDGES = 320000
N_GRAPHS = 64

NC, NS, LANES = 2, 16, 16
NW = NC * NS
CHUNK = 48
NCHUNK = 216
EPW = CHUNK * NCHUNK
PAD_DST = 10239
ROWS_PER_SUB = N_PAD // NS
ZROWS = 128


def _bf16(x):
    return x.astype(jnp.bfloat16)


def _rne16(x):
    u = lax.bitcast_convert_type(x, jnp.int32)
    r = u + jnp.int32(0x7FFF) + ((u >> 16) & 1)
    r = r & jnp.int32(-65536)
    return lax.bitcast_convert_type(r, jnp.float32)


def _edge_pass(D):
    mesh = plsc.VectorSubcoreMesh(core_axis_name="c", subcore_axis_name="s")

    @functools.partial(
        pl.kernel,
        out_type=(
            jax.ShapeDtypeStruct((NC, N_PAD, D), jnp.float32),
            jax.ShapeDtypeStruct((NC, N_PAD), jnp.float32),
        ),
        mesh=mesh,
        compiler_params=pltpu.CompilerParams(needs_layout_passes=False,
                                             use_tc_tiling_on_sc=False),
        scratch_types=(
            pltpu.VMEM_SHARED((N_PAD, D), jnp.float32),
            pltpu.VMEM_SHARED((N_PAD,), jnp.float32),
            pltpu.VMEM((2, CHUNK), jnp.int32),
            pltpu.VMEM((2, CHUNK), jnp.int32),
            pltpu.VMEM((2, CHUNK), jnp.int32),
            pltpu.VMEM((2, CHUNK), jnp.int32),
            pltpu.VMEM((CHUNK, D), jnp.float32),
            pltpu.VMEM((CHUNK, D), jnp.float32),
            pltpu.VMEM((CHUNK, D), jnp.float32),
            pltpu.VMEM((CHUNK, D), jnp.float32),
            pltpu.VMEM((CHUNK, D), jnp.float32),
            pltpu.VMEM((CHUNK, D), jnp.float32),
            pltpu.VMEM((CHUNK,), jnp.float32),
            pltpu.VMEM((CHUNK,), jnp.float32),
            pltpu.VMEM((D,), jnp.float32),
            pltpu.SemaphoreType.DMA,
            pltpu.SemaphoreType.DMA,
            pltpu.SemaphoreType.DMA,
            pltpu.SemaphoreType.DMA,
            pltpu.SemaphoreType.DMA,
            pltpu.SemaphoreType.DMA,
            pltpu.SemaphoreType.DMA,
            pltpu.SemaphoreType.DMA,
        ),
    )
    def edge_kernel(xl_hbm, xr_hbm, a_hbm, edges_hbm, znum_hbm,
                    zden_hbm, num_out, den_out,
                    num_sh, den_sh, idx0, idx1, idx2, idx3,
                    rs0, rs1, rd0, rd1, wv0, wv1, wb0, wb1, av,
                    si0, si1, si2, si3, sg0, sg1, ss0, ss1):
        idx = [idx0, idx1, idx2, idx3]
        rs = [rs0, rs1]
        rd = [rd0, rd1]
        wv = [wv0, wv1]
        wb = [wb0, wb1]
        si = [si0, si1, si2, si3]
        sg = [sg0, sg1]
        ss = [ss0, ss1]

        cid = lax.axis_index("c")
        sid = lax.axis_index("s")
        wid = sid * NC + cid
        base = sid * ROWS_PER_SUB
        nsl = pl.ds(base, ROWS_PER_SUB)

        pltpu.sync_copy(znum_hbm.at[nsl], num_sh.at[nsl])
        pltpu.sync_copy(zden_hbm.at[nsl], den_sh.at[nsl])
        pltpu.sync_copy(a_hbm, av)

        plsc.subcore_barrier()

        slope = np.float32(0.2)
        eidx0 = jnp.arange(LANES, dtype=jnp.int32)
        av_regs = [_rne16(av[pl.ds(i * LANES, LANES)])
                   for i in range(D // LANES)]

        def idx_start(islot, j):
            pltpu.async_copy(edges_hbm.at[wid, j], idx[islot], si[islot])

        def idx_wait(islot, j):
            pltpu.make_async_copy(edges_hbm.at[wid, j], idx[islot],
                                  si[islot]).wait()

        def gather_start(s, islot):
            pltpu.async_copy(xl_hbm.at[idx[islot].at[0]], rs[s], sg[s])
            pltpu.async_copy(xr_hbm.at[idx[islot].at[1]], rd[s], sg[s])

        def gather_wait(s, islot):
            pltpu.make_async_copy(xl_hbm.at[idx[islot].at[0]], rs[s],
                                  sg[s]).wait()
            pltpu.make_async_copy(xr_hbm.at[idx[islot].at[1]], rd[s],
                                  sg[s]).wait()

        def scatter_start(s, islot):
            pltpu.async_copy(wv[s], num_sh.at[idx[islot].at[1]], ss[s],
                             add=True)
            pltpu.async_copy(wb[s], den_sh.at[idx[islot].at[1]], ss[s],
                             add=True)

        def scatter_wait(s, islot):
            pltpu.make_async_copy(wv[s], num_sh.at[idx[islot].at[1]],
                                  ss[s]).wait()
            pltpu.make_async_copy(wb[s], den_sh.at[idx[islot].at[1]],
                                  ss[s]).wait()

        def compute(s):
            def group_body(g, _):
                lvec = jnp.zeros((LANES,), jnp.float32)
                for i in range(LANES):
                    e = g * LANES + i
                    acc = jnp.zeros((LANES,), jnp.float32)
                    for k in range(D // LANES):
                        sl = pl.ds(k * LANES, LANES)
                        v = rs[s][e, sl] + rd[s][e, sl]
                        lr = _rne16(jnp.maximum(v, v * slope))
                        acc = acc + lr * av_regs[k]
                    lvec = jnp.where(eidx0 == i, jnp.sum(acc), lvec)
                wvec = jnp.exp(lvec)
                wb[s][pl.ds(g * LANES, LANES)] = wvec
                for i in range(LANES):
                    e = g * LANES + i
                    for k in range(D // LANES):
                        sl = pl.ds(k * LANES, LANES)
                        wv[s][e, sl] = rs[s][e, sl] * wvec[i]
                return 0
            lax.fori_loop(0, CHUNK // LANES, group_body, 0)

        def body(j, jm4, jm2, warm, has_next, has_next2):
            s = jm2
            islot = jm4
            if not warm:
                scatter_wait(s, (jm4 + 2) % 4)
            if has_next2:
                idx_start((jm4 + 2) % 4, j + 2)
            if has_next:
                idx_wait((jm4 + 1) % 4, j + 1)
                gather_start(1 - s, (jm4 + 1) % 4)
            gather_wait(s, islot)
            compute(s)
            scatter_start(s, islot)

        idx_start(0, 0)
        idx_start(1, 1)
        idx_wait(0, 0)
        gather_start(0, 0)
        body(0, 0, 0, True, True, True)
        body(1, 1, 1, True, True, True)

        def mid_body(i, _):
            j = 4 * i + 2
            for b in range(4):
                body(j + b, (2 + b) % 4, b % 2, False, True, True)
            return 0
        lax.fori_loop(0, (NCHUNK - 4) // 4, mid_body, 0)

        body(NCHUNK - 2, (NCHUNK - 2) % 4, (NCHUNK - 2) % 2, False, True,
             False)
        body(NCHUNK - 1, (NCHUNK - 1) % 4, (NCHUNK - 1) % 2, False, False,
             False)
        scatter_wait((NCHUNK - 2) % 2, (NCHUNK - 2) % 4)
        scatter_wait((NCHUNK - 1) % 2, (NCHUNK - 1) % 4)

        plsc.subcore_barrier()

        pltpu.sync_copy(num_sh.at[nsl], num_out.at[cid].at[nsl])
        pltpu.sync_copy(den_sh.at[nsl], den_out.at[cid].at[nsl])

    return edge_kernel


_EDGE_KERNELS = {d: _edge_pass(d) for d in (32, 64, 128)}


def _pre_body(x_ref, wl_ref, wr_ref, xl_ref, xr_ref):
    x = x_ref[...]
    xl_ref[:N_NODES, :] = jnp.dot(_bf16(x), _bf16(wl_ref[...]),
                                  preferred_element_type=jnp.float32)
    xr_ref[:N_NODES, :] = jnp.dot(_bf16(x), _bf16(wr_ref[...]),
                                  preferred_element_type=jnp.float32)
    pad = jnp.zeros((N_PAD - N_NODES, wl_ref.shape[1]), jnp.float32)
    xl_ref[N_NODES:, :] = pad
    xr_ref[N_NODES:, :] = pad


def _mid_body(num_ref, den_ref, b_ref, wl_ref, wr_ref, xl_ref, xr_ref):
    num = num_ref[0, :N_NODES, :] + num_ref[1, :N_NODES, :]
    den = den_ref[0, :N_NODES, :] + den_ref[1, :N_NODES, :]
    h = num / (den + np.float32(1e-16)) + b_ref[...]
    h = jnp.maximum(h, h * np.float32(0.01))
    xl_ref[:N_NODES, :] = jnp.dot(_bf16(h), _bf16(wl_ref[...]),
                                  preferred_element_type=jnp.float32)
    xr_ref[:N_NODES, :] = jnp.dot(_bf16(h), _bf16(wr_ref[...]),
                                  preferred_element_type=jnp.float32)
    pad = jnp.zeros((N_PAD - N_NODES, wl_ref.shape[1]), jnp.float32)
    xl_ref[N_NODES:, :] = pad
    xr_ref[N_NODES:, :] = pad


def _final_body(num_ref, den_ref, b_ref, batch_ref,
                wc1_ref, bc1_ref, wc2_ref, bc2_ref, wc3_ref, bc3_ref,
                wc4_ref, bc4_ref, wc5_ref, bc5_ref, z_ref, g_ref):
    num = num_ref[0, :N_NODES, :] + num_ref[1, :N_NODES, :]
    den = den_ref[0, :N_NODES, :] + den_ref[1, :N_NODES, :]
    h = num / (den + np.float32(1e-16)) + b_ref[...]
    bvec = batch_ref[...]

    def pool_body(g, _):
        hm = jnp.where(bvec == g, h, -jnp.inf)
        g_ref[pl.ds(g, 1), :] = jnp.max(hm, axis=0, keepdims=True)
        return 0
    lax.fori_loop(0, N_GRAPHS, pool_body, 0)

    z = g_ref[...]
    z = jnp.maximum(jnp.dot(_bf16(z), _bf16(wc1_ref[...]), preferred_element_type=jnp.float32) + bc1_ref[...], 0.0)
    z = jnp.maximum(jnp.dot(_bf16(z), _bf16(wc2_ref[...]), preferred_element_type=jnp.float32) + bc2_ref[...], 0.0)
    z = jnp.maximum(jnp.dot(_bf16(z), _bf16(wc3_ref[...]), preferred_element_type=jnp.float32) + bc3_ref[...], 0.0)
    z = jnp.maximum(jnp.dot(_bf16(z), _bf16(wc4_ref[...]), preferred_element_type=jnp.float32) + bc4_ref[...], 0.0)
    z_ref[...] = jnp.dot(_bf16(z), _bf16(wc5_ref[...]), preferred_element_type=jnp.float32) + bc5_ref[...]


def _pre_call(x, wl, wr):
    dout = wl.shape[1]
    return pl.pallas_call(
        _pre_body,
        out_shape=(jax.ShapeDtypeStruct((N_PAD, dout), jnp.float32),
                   jax.ShapeDtypeStruct((N_PAD, dout), jnp.float32)),
    )(x, wl, wr)


def _mid_call(num, den, b, wl, wr):
    dout = wl.shape[1]
    return pl.pallas_call(
        _mid_body,
        out_shape=(jax.ShapeDtypeStruct((N_PAD, dout), jnp.float32),
                   jax.ShapeDtypeStruct((N_PAD, dout), jnp.float32)),
    )(num, den.reshape(NC, N_PAD, 1), b.reshape(1, -1), wl, wr)


def _final_call(num, den, b, batch, wc_bc):
    args = [num, den.reshape(NC, N_PAD, 1), b.reshape(1, -1),
            batch.reshape(N_NODES, 1)]
    for w, bc in wc_bc:
        args.extend([w, bc.reshape(1, -1)])
    return pl.pallas_call(
        _final_body,
        out_shape=jax.ShapeDtypeStruct((N_GRAPHS, 4), jnp.float32),
        scratch_shapes=[pltpu.VMEM((N_GRAPHS, 128), jnp.float32)],
    )(*args)


def kernel(x, edge_index, batch,
           Wl1, Wr1, a1, b1,
           Wl2, Wr2, a2, b2,
           Wl3, Wr3, a3, b3,
           Wc1, bc1, Wc2, bc2, Wc3, bc3, Wc4, bc4, Wc5, bc5):
    ppw = EPW - N_EDGES // NW
    pad_src = jnp.broadcast_to(jnp.arange(ppw, dtype=jnp.int32), (NW, ppw))
    pad_dst = jnp.broadcast_to(
        N_NODES + (jnp.arange(ppw, dtype=jnp.int32) % (N_PAD - N_NODES)),
        (NW, ppw))
    srcp = jnp.concatenate(
        [edge_index[0].reshape(NW, -1), pad_src], axis=1).reshape(NW, NCHUNK, CHUNK)
    dstp = jnp.concatenate(
        [edge_index[1].reshape(NW, -1), pad_dst], axis=1).reshape(NW, NCHUNK, CHUNK)
    edges = jnp.stack([srcp, dstp], axis=2)

    zden = jnp.zeros((N_PAD,), jnp.float32)

    xl, xr = _pre_call(x, Wl1, Wr1)
    num, den = _EDGE_KERNELS[32](xl, xr, a1, edges,
                                 jnp.zeros((N_PAD, 32), jnp.float32), zden)
    xl, xr = _mid_call(num, den, b1, Wl2, Wr2)
    num, den = _EDGE_KERNELS[64](xl, xr, a2, edges,
                                 jnp.zeros((N_PAD, 64), jnp.float32), zden)
    xl, xr = _mid_call(num, den, b2, Wl3, Wr3)
    num, den = _EDGE_KERNELS[128](xl, xr, a3, edges,
                                  jnp.zeros((N_PAD, 128), jnp.float32), zden)
    return _final_call(num, den, b3, batch,
                       [(Wc1, bc1), (Wc2, bc2), (Wc3, bc3), (Wc4, bc4), (Wc5, bc5)])


# speedup vs baseline: 19.9223x; 1.0011x over previous
import functools

import jax
import jax.numpy as jnp
import numpy as np
from jax import lax
from jax.experimental import pallas as pl
from jax.experimental.pallas import tpu as pltpu
from jax.experimental.pallas import tpu_sc as plsc

N_NODES = 10000
N_PAD = 10240
N_EDGES = 320000
N_GRAPHS = 64

NC, NS, LANES = 2, 16, 16
NW = NC * NS
CHUNK = 48
NCHUNK = 216
EPW = CHUNK * NCHUNK
ROWS_PER_SUB = N_PAD // NS


def _bf16(x):
    return x.astype(jnp.bfloat16)


def _rne16(x):
    u = lax.bitcast_convert_type(x, jnp.int32)
    r = u + jnp.int32(0x7FFF) + ((u >> 16) & 1)
    r = r & jnp.int32(-65536)
    return lax.bitcast_convert_type(r, jnp.float32)


def _edge_pass(D):
    mesh = plsc.VectorSubcoreMesh(core_axis_name="c", subcore_axis_name="s",
                                  num_cores=NC, num_subcores=NS)

    @functools.partial(
        pl.kernel,
        out_type=(
            jax.ShapeDtypeStruct((NC, N_PAD, D), jnp.float32),
            jax.ShapeDtypeStruct((NC, N_PAD), jnp.float32),
        ),
        mesh=mesh,
        compiler_params=pltpu.CompilerParams(needs_layout_passes=False,
                                             use_tc_tiling_on_sc=False),
        scratch_types=(
            pltpu.VMEM_SHARED((N_PAD, D), jnp.float32),
            pltpu.VMEM_SHARED((N_PAD,), jnp.float32),
            pltpu.VMEM((2, CHUNK), jnp.int32),
            pltpu.VMEM((2, CHUNK), jnp.int32),
            pltpu.VMEM((2, CHUNK), jnp.int32),
            pltpu.VMEM((2, CHUNK), jnp.int32),
            pltpu.VMEM((CHUNK, D), jnp.float32),
            pltpu.VMEM((CHUNK, D), jnp.float32),
            pltpu.VMEM((CHUNK, D), jnp.float32),
            pltpu.VMEM((CHUNK, D), jnp.float32),
            pltpu.VMEM((CHUNK, D), jnp.float32),
            pltpu.VMEM((CHUNK, D), jnp.float32),
            pltpu.VMEM((CHUNK,), jnp.float32),
            pltpu.VMEM((CHUNK,), jnp.float32),
            pltpu.VMEM((D,), jnp.float32),
            pltpu.SemaphoreType.DMA,
            pltpu.SemaphoreType.DMA,
            pltpu.SemaphoreType.DMA,
            pltpu.SemaphoreType.DMA,
            pltpu.SemaphoreType.DMA,
            pltpu.SemaphoreType.DMA,
            pltpu.SemaphoreType.DMA,
            pltpu.SemaphoreType.DMA,
        ),
    )
    def edge_kernel(xl_hbm, xr_hbm, a_hbm, edges_hbm, znum_hbm,
                    zden_hbm, num_out, den_out,
                    num_sh, den_sh, idx0, idx1, idx2, idx3,
                    rs0, rs1, rd0, rd1, wv0, wv1, wb0, wb1, av,
                    si0, si1, si2, si3, sg0, sg1, ss0, ss1):
        idx = [idx0, idx1, idx2, idx3]
        rs = [rs0, rs1]
        rd = [rd0, rd1]
        wv = [wv0, wv1]
        wb = [wb0, wb1]
        si = [si0, si1, si2, si3]
        sg = [sg0, sg1]
        ss = [ss0, ss1]

        cid = lax.axis_index("c")
        sid = lax.axis_index("s")
        wid = sid * NC + cid
        base = sid * ROWS_PER_SUB
        nsl = pl.ds(base, ROWS_PER_SUB)

        pltpu.sync_copy(znum_hbm.at[nsl], num_sh.at[nsl])
        pltpu.sync_copy(zden_hbm.at[nsl], den_sh.at[nsl])
        pltpu.sync_copy(a_hbm, av)

        plsc.subcore_barrier()

        slope = np.float32(0.2)
        eidx0 = jnp.arange(LANES, dtype=jnp.int32)
        av_regs = [_rne16(av[pl.ds(i * LANES, LANES)])
                   for i in range(D // LANES)]

        def idx_start(islot, j):
            pltpu.async_copy(edges_hbm.at[wid, j], idx[islot], si[islot])

        def idx_wait(islot, j):
            pltpu.make_async_copy(edges_hbm.at[wid, j], idx[islot],
                                  si[islot]).wait()

        def gather_start(s, islot):
            pltpu.async_copy(xl_hbm.at[idx[islot].at[0]], rs[s], sg[s])
            pltpu.async_copy(xr_hbm.at[idx[islot].at[1]], rd[s], sg[s])

        def gather_wait(s, islot):
            pltpu.make_async_copy(xl_hbm.at[idx[islot].at[0]], rs[s],
                                  sg[s]).wait()
            pltpu.make_async_copy(xr_hbm.at[idx[islot].at[1]], rd[s],
                                  sg[s]).wait()

        def scatter_start(s, islot):
            pltpu.async_copy(wv[s], num_sh.at[idx[islot].at[1]], ss[s],
                             add=True)
            pltpu.async_copy(wb[s], den_sh.at[idx[islot].at[1]], ss[s],
                             add=True)

        def scatter_wait(s, islot):
            pltpu.make_async_copy(wv[s], num_sh.at[idx[islot].at[1]],
                                  ss[s]).wait()
            pltpu.make_async_copy(wb[s], den_sh.at[idx[islot].at[1]],
                                  ss[s]).wait()

        def compute(s):
            def group_body(g, _):
                lvec = jnp.zeros((LANES,), jnp.float32)
                for i in range(LANES):
                    e = g * LANES + i
                    acc = jnp.zeros((LANES,), jnp.float32)
                    for k in range(D // LANES):
                        sl = pl.ds(k * LANES, LANES)
                        v = rs[s][e, sl] + rd[s][e, sl]
                        lr = _rne16(jnp.maximum(v, v * slope))
                        acc = acc + lr * av_regs[k]
                    lvec = jnp.where(eidx0 == i, jnp.sum(acc), lvec)
                wvec = jnp.exp(lvec)
                wb[s][pl.ds(g * LANES, LANES)] = wvec
                for i in range(LANES):
                    e = g * LANES + i
                    for k in range(D // LANES):
                        sl = pl.ds(k * LANES, LANES)
                        wv[s][e, sl] = rs[s][e, sl] * wvec[i]
                return 0
            lax.fori_loop(0, CHUNK // LANES, group_body, 0)

        def body(j, jm4, jm2, warm, has_next, has_next2):
            s = jm2
            islot = jm4
            if not warm:
                scatter_wait(s, (jm4 + 2) % 4)
            if has_next2:
                idx_start((jm4 + 2) % 4, j + 2)
            if has_next:
                idx_wait((jm4 + 1) % 4, j + 1)
                gather_start(1 - s, (jm4 + 1) % 4)
            gather_wait(s, islot)
            compute(s)
            scatter_start(s, islot)

        idx_start(0, 0)
        idx_start(1, 1)
        idx_wait(0, 0)
        gather_start(0, 0)
        body(0, 0, 0, True, True, True)
        body(1, 1, 1, True, True, True)

        def mid_body(i, _):
            j = 4 * i + 2
            for b in range(4):
                body(j + b, (2 + b) % 4, b % 2, False, True, True)
            return 0
        lax.fori_loop(0, (NCHUNK - 4) // 4, mid_body, 0)

        body(NCHUNK - 2, (NCHUNK - 2) % 4, (NCHUNK - 2) % 2, False, True,
             False)
        body(NCHUNK - 1, (NCHUNK - 1) % 4, (NCHUNK - 1) % 2, False, False,
             False)
        scatter_wait((NCHUNK - 2) % 2, (NCHUNK - 2) % 4)
        scatter_wait((NCHUNK - 1) % 2, (NCHUNK - 1) % 4)

        plsc.subcore_barrier()

        pltpu.sync_copy(num_sh.at[nsl], num_out.at[cid].at[nsl])
        pltpu.sync_copy(den_sh.at[nsl], den_out.at[cid].at[nsl])

    return edge_kernel


_EDGE_KERNELS = {}


def _edge_kernel(d):
    if d not in _EDGE_KERNELS:
        _EDGE_KERNELS[d] = _edge_pass(d)
    return _EDGE_KERNELS[d]


def _pre_body(x_ref, wl_ref, wr_ref, xl_ref, xr_ref):
    x = x_ref[...]
    xl_ref[:N_NODES, :] = jnp.dot(_bf16(x), _bf16(wl_ref[...]),
                                  preferred_element_type=jnp.float32)
    xr_ref[:N_NODES, :] = jnp.dot(_bf16(x), _bf16(wr_ref[...]),
                                  preferred_element_type=jnp.float32)
    pad = jnp.zeros((N_PAD - N_NODES, wl_ref.shape[1]), jnp.float32)
    xl_ref[N_NODES:, :] = pad
    xr_ref[N_NODES:, :] = pad


def _mid_body(num_ref, den_ref, b_ref, wl_ref, wr_ref, xl_ref, xr_ref):
    num = num_ref[0, :N_NODES, :] + num_ref[1, :N_NODES, :]
    den = den_ref[0, :N_NODES, :] + den_ref[1, :N_NODES, :]
    h = num / (den + np.float32(1e-16)) + b_ref[...]
    h = jnp.maximum(h, h * np.float32(0.01))
    xl_ref[:N_NODES, :] = jnp.dot(_bf16(h), _bf16(wl_ref[...]),
                                  preferred_element_type=jnp.float32)
    xr_ref[:N_NODES, :] = jnp.dot(_bf16(h), _bf16(wr_ref[...]),
                                  preferred_element_type=jnp.float32)
    pad = jnp.zeros((N_PAD - N_NODES, wl_ref.shape[1]), jnp.float32)
    xl_ref[N_NODES:, :] = pad
    xr_ref[N_NODES:, :] = pad


def _final_body(num_ref, den_ref, b_ref, batch_ref,
                wc1_ref, bc1_ref, wc2_ref, bc2_ref, wc3_ref, bc3_ref,
                wc4_ref, bc4_ref, wc5_ref, bc5_ref, z_ref, g_ref):
    num = num_ref[0, :N_NODES, :] + num_ref[1, :N_NODES, :]
    den = den_ref[0, :N_NODES, :] + den_ref[1, :N_NODES, :]
    h = num / (den + np.float32(1e-16)) + b_ref[...]
    bvec = batch_ref[...]

    def pool_body(g, _):
        hm = jnp.where(bvec == g, h, -jnp.inf)
        g_ref[pl.ds(g, 1), :] = jnp.max(hm, axis=0, keepdims=True)
        return 0
    lax.fori_loop(0, N_GRAPHS, pool_body, 0)

    z = g_ref[...]
    z = jnp.maximum(jnp.dot(_bf16(z), _bf16(wc1_ref[...]), preferred_element_type=jnp.float32) + bc1_ref[...], 0.0)
    z = jnp.maximum(jnp.dot(_bf16(z), _bf16(wc2_ref[...]), preferred_element_type=jnp.float32) + bc2_ref[...], 0.0)
    z = jnp.maximum(jnp.dot(_bf16(z), _bf16(wc3_ref[...]), preferred_element_type=jnp.float32) + bc3_ref[...], 0.0)
    z = jnp.maximum(jnp.dot(_bf16(z), _bf16(wc4_ref[...]), preferred_element_type=jnp.float32) + bc4_ref[...], 0.0)
    z_ref[...] = jnp.dot(_bf16(z), _bf16(wc5_ref[...]), preferred_element_type=jnp.float32) + bc5_ref[...]


def _pre_call(x, wl, wr):
    dout = wl.shape[1]
    return pl.pallas_call(
        _pre_body,
        out_shape=(jax.ShapeDtypeStruct((N_PAD, dout), jnp.float32),
                   jax.ShapeDtypeStruct((N_PAD, dout), jnp.float32)),
    )(x, wl, wr)


def _mid_call(num, den, b, wl, wr):
    dout = wl.shape[1]
    return pl.pallas_call(
        _mid_body,
        out_shape=(jax.ShapeDtypeStruct((N_PAD, dout), jnp.float32),
                   jax.ShapeDtypeStruct((N_PAD, dout), jnp.float32)),
    )(num, den.reshape(NC, N_PAD, 1), b.reshape(1, -1), wl, wr)


def _final_call(num, den, b, batch, wc_bc):
    args = [num, den.reshape(NC, N_PAD, 1), b.reshape(1, -1),
            batch.reshape(N_NODES, 1)]
    for w, bc in wc_bc:
        args.extend([w, bc.reshape(1, -1)])
    return pl.pallas_call(
        _final_body,
        out_shape=jax.ShapeDtypeStruct((N_GRAPHS, 4), jnp.float32),
        scratch_shapes=[pltpu.VMEM((N_GRAPHS, 128), jnp.float32)],
    )(*args)


def kernel(x, edge_index, batch,
           Wl1, Wr1, a1, b1,
           Wl2, Wr2, a2, b2,
           Wl3, Wr3, a3, b3,
           Wc1, bc1, Wc2, bc2, Wc3, bc3, Wc4, bc4, Wc5, bc5):
    ppw = EPW - N_EDGES // NW
    pad_src = jnp.broadcast_to(jnp.arange(ppw, dtype=jnp.int32), (NW, ppw))
    pad_dst = jnp.broadcast_to(
        N_NODES + (jnp.arange(ppw, dtype=jnp.int32) % (N_PAD - N_NODES)),
        (NW, ppw))
    srcp = jnp.concatenate(
        [edge_index[0].reshape(NW, -1), pad_src], axis=1).reshape(NW, NCHUNK, CHUNK)
    dstp = jnp.concatenate(
        [edge_index[1].reshape(NW, -1), pad_dst], axis=1).reshape(NW, NCHUNK, CHUNK)
    edges = jnp.stack([srcp, dstp], axis=2)

    zden = jnp.zeros((N_PAD,), jnp.float32)

    xl, xr = _pre_call(x, Wl1, Wr1)
    num, den = _edge_kernel(32)(xl, xr, a1, edges,
                                 jnp.zeros((N_PAD, 32), jnp.float32), zden)
    xl, xr = _mid_call(num, den, b1, Wl2, Wr2)
    num, den = _edge_kernel(64)(xl, xr, a2, edges,
                                 jnp.zeros((N_PAD, 64), jnp.float32), zden)
    xl, xr = _mid_call(num, den, b2, Wl3, Wr3)
    num, den = _edge_kernel(128)(xl, xr, a3, edges,
                                  jnp.zeros((N_PAD, 128), jnp.float32), zden)
    return _final_call(num, den, b3, batch,
                       [(Wc1, bc1), (Wc2, bc2), (Wc3, bc3), (Wc4, bc4), (Wc5, bc5)])
